# baseline (speedup 1.0000x reference)
import functools

import jax
import jax.numpy as jnp
from jax import lax
from jax.experimental import pallas as pl
from jax.experimental.pallas import tpu as pltpu
from jax.experimental.pallas import tpu_sc as plsc

_B = 16384
_D = 32
_NC = 2
_NS = 16
_NW = _NC * _NS
_BPW = _B // _NW

_mesh = plsc.VectorSubcoreMesh(core_axis_name="c", subcore_axis_name="s")


@functools.partial(
    pl.kernel,
    out_type=(
        jax.ShapeDtypeStruct((_B, _D), jnp.float32),
        jax.ShapeDtypeStruct((_B, _D), jnp.float32),
        jax.ShapeDtypeStruct((_B, _D), jnp.float32),
    ),
    mesh=_mesh,
    compiler_params=pltpu.CompilerParams(use_tc_tiling_on_sc=False),
    scratch_types=[
        pltpu.VMEM((_BPW,), jnp.int32),
        pltpu.VMEM((_BPW,), jnp.int32),
        pltpu.VMEM((_BPW,), jnp.int32),
        pltpu.VMEM((_BPW, _D), jnp.float32),
        pltpu.VMEM((_BPW, _D), jnp.float32),
        pltpu.VMEM((_BPW, _D), jnp.float32),
        pltpu.SemaphoreType.DMA,
        pltpu.SemaphoreType.DMA,
        pltpu.SemaphoreType.DMA,
    ],
)
def _gather3(src_t, edge_t, dst_t, src_i, edge_i, dst_i,
             src_o, edge_o, dst_o,
             i0, i1, i2, r0, r1, r2, s0, s1, s2):
    wid = lax.axis_index("s") * _NC + lax.axis_index("c")
    base = wid * _BPW
    pltpu.sync_copy(src_i.at[pl.ds(base, _BPW)], i0)
    pltpu.sync_copy(edge_i.at[pl.ds(base, _BPW)], i1)
    pltpu.sync_copy(dst_i.at[pl.ds(base, _BPW)], i2)
    c0 = pltpu.async_copy(src_t.at[i0], r0, s0)
    c1 = pltpu.async_copy(edge_t.at[i1], r1, s1)
    c2 = pltpu.async_copy(dst_t.at[i2], r2, s2)
    c0.wait()
    pltpu.sync_copy(r0, src_o.at[pl.ds(base, _BPW)])
    c1.wait()
    pltpu.sync_copy(r1, edge_o.at[pl.ds(base, _BPW)])
    c2.wait()
    pltpu.sync_copy(r2, dst_o.at[pl.ds(base, _BPW)])


def kernel(src_table, edge_table, dst_table, src_ids, edge_ids, dst_ids, offset):
    src_emb, edge_emb, dst_emb = _gather3(
        src_table, edge_table, dst_table,
        src_ids.astype(jnp.int32),
        edge_ids.astype(jnp.int32),
        dst_ids.astype(jnp.int32),
    )
    return (src_emb, edge_emb, dst_emb, offset)


# speedup vs baseline: 1.3371x; 1.3371x over previous
import functools

import jax
import jax.numpy as jnp
from jax import lax
from jax.experimental import pallas as pl
from jax.experimental.pallas import tpu as pltpu
from jax.experimental.pallas import tpu_sc as plsc

_B = 16384
_D = 32
_NC = 2
_NS = 16
_NW = _NC * _NS
_BPW = _B // _NW
_CH = 16

_mesh = plsc.VectorSubcoreMesh(core_axis_name="c", subcore_axis_name="s")


@functools.partial(
    pl.kernel,
    out_type=(
        jax.ShapeDtypeStruct((_B, _D), jnp.float32),
        jax.ShapeDtypeStruct((_B, _D), jnp.float32),
        jax.ShapeDtypeStruct((_B, _D), jnp.float32),
    ),
    mesh=_mesh,
    scratch_types=[
        pltpu.SMEM((_BPW,), jnp.int32),
        pltpu.VMEM((_BPW,), jnp.int32),
        pltpu.VMEM((_BPW, _D), jnp.float32),
        pltpu.SemaphoreType.DMA,
        pltpu.SemaphoreType.DMA,
    ],
)
def _gather3(src_t, edge_t, dst_t, src_i, edge_i, dst_i,
             src_o, edge_o, dst_o,
             idx_s, idx_v, rows_v, sem, osem):
    wid = lax.axis_index("s") * _NC + lax.axis_index("c")
    base = wid * _BPW

    def one_table(table, ids, out):
        pltpu.sync_copy(ids.at[pl.ds(base, _BPW)], idx_v)

        @pl.loop(0, _BPW // _CH)
        def chunk(c):
            cbase = c * _CH
            vec = idx_v[pl.ds(cbase, _CH)]
            copies = []
            for j in range(_CH):
                row = vec[j]
                copies.append(
                    pltpu.async_copy(table.at[row], rows_v.at[cbase + j], sem)
                )
            for cp in copies:
                cp.wait()
        pltpu.async_copy(rows_v, out.at[pl.ds(base, _BPW)], osem).wait()

    one_table(src_t, src_i, src_o)
    one_table(edge_t, edge_i, edge_o)
    one_table(dst_t, dst_i, dst_o)


def kernel(src_table, edge_table, dst_table, src_ids, edge_ids, dst_ids, offset):
    src_emb, edge_emb, dst_emb = _gather3(
        src_table, edge_table, dst_table,
        src_ids.astype(jnp.int32),
        edge_ids.astype(jnp.int32),
        dst_ids.astype(jnp.int32),
    )
    return (src_emb, edge_emb, dst_emb, offset)
